# Initial kernel scaffold; baseline (speedup 1.0000x reference)
import jax
import jax.numpy as jnp
from jax.experimental import pallas as pl


def kernel(occ, params):
    raise NotImplementedError("write your pallas kernel here")


# speedup vs baseline: 239.4368x; 239.4368x over previous
import jax
import jax.numpy as jnp
from jax.experimental import pallas as pl

_M = 64
_N3 = _M * _M * _M
_FN = 512
_FD = 32
_LAT = 33


def _conv_mm_kernel(x_ref, w_ref, b_ref, o_ref):
    o_ref[0] = jnp.maximum(
        jnp.dot(w_ref[...], x_ref[0], preferred_element_type=jnp.float32, precision=jax.lax.Precision.HIGHEST)
        + b_ref[...], 0.0)


def _conv3_pool_kernel(x_ref, w_ref, b_ref, occ_ref, o_ref):
    h = jnp.maximum(
        jnp.dot(w_ref[...], x_ref[0], preferred_element_type=jnp.float32, precision=jax.lax.Precision.HIGHEST)
        + b_ref[...], 0.0)
    mask = jnp.max(occ_ref[0], axis=1, keepdims=True)
    denom = jnp.maximum(jnp.sum(mask), 1.0)
    o_ref[0] = jnp.dot(h, mask, preferred_element_type=jnp.float32, precision=jax.lax.Precision.HIGHEST) / denom


def _fc_kernel(w_ref, p_ref, b_ref, o_ref):
    o_ref[...] = jnp.dot(w_ref[...], p_ref[...],
                         preferred_element_type=jnp.float32, precision=jax.lax.Precision.HIGHEST) + b_ref[...]


def _latent_kernel(x_ref, eps_ref, muw_ref, mub_ref, lvw_ref, lvb_ref,
                   dw_ref, db_ref, d1w_ref, a_ref, kl_ref):
    x = x_ref[0]
    mu = jnp.dot(x, muw_ref[...], preferred_element_type=jnp.float32) + mub_ref[...]
    lv = jnp.clip(
        jnp.dot(x, lvw_ref[...], preferred_element_type=jnp.float32) + lvb_ref[...],
        -30.0, 20.0)
    elv = jnp.exp(lv)
    std = jnp.exp(0.5 * lv)
    z = mu + std * eps_ref[0]
    kl_ref[0] = 0.5 * jnp.sum(mu * mu + elv - 1.0 - lv) * jnp.ones((1, 1), jnp.float32)

    cq = (jax.lax.broadcasted_iota(jnp.int32, (_M, 1), 0).astype(jnp.float32)
          + 0.5) / _M
    lanes = jax.lax.broadcasted_iota(jnp.int32, (_M, _FN), 1)
    rr = jax.lax.broadcasted_iota(jnp.int32, (_FN - 1, _FN), 0)
    cc = jax.lax.broadcasted_iota(jnp.int32, (_FN - 1, _FN), 1)
    utri = (rr < cc).astype(jnp.float32)

    for d in range(3):
        zd = z[d * _FN:(d + 1) * _FN]
        fd = zd[:, :_FD]
        dcol = zd[:, _FD:_FD + 1]
        draw = jax.lax.dot_general(
            dcol, dw_ref[...], (((0,), (0,)), ((), ())),
            preferred_element_type=jnp.float32) + db_ref[...]
        mx = jnp.max(draw, axis=1, keepdims=True)
        ex = jnp.exp(draw - mx)
        sp = ex / jnp.sum(ex, axis=1, keepdims=True)
        pos = jnp.dot(sp, utri, preferred_element_type=jnp.float32, precision=jax.lax.Precision.HIGHEST)
        cnt = jnp.sum((pos <= cq).astype(jnp.int32), axis=1, keepdims=True)
        idx = jnp.clip(cnt - 1, 0, _FN - 2)
        oh0 = (lanes == idx).astype(jnp.float32)
        oh1 = (lanes == (idx + 1)).astype(jnp.float32)
        p0 = jnp.sum(oh0 * pos, axis=1, keepdims=True)
        p1 = jnp.sum(oh1 * pos, axis=1, keepdims=True)
        t = jnp.clip((cq - p0) / (p1 - p0 + 1e-8), 0.0, 1.0)
        l0 = jnp.dot(oh0, fd, preferred_element_type=jnp.float32, precision=jax.lax.Precision.HIGHEST)
        l1 = jnp.dot(oh1, fd, preferred_element_type=jnp.float32, precision=jax.lax.Precision.HIGHEST)
        ld = l0 * (1.0 - t) + l1 * t
        a_ref[0, d] = jnp.dot(ld, d1w_ref[d * _FD:(d + 1) * _FD],
                              preferred_element_type=jnp.float32, precision=jax.lax.Precision.HIGHEST)


def _decode_kernel(a_ref, b1_ref, w2_ref, o_ref):
    i = pl.program_id(1)
    s = a_ref[0, 0, pl.ds(i, 1), :] + b1_ref[...]
    ay = a_ref[0, 1]
    az = a_ref[0, 2]
    tmp = ay[:, None, :] + az[None, :, :] + s[None, :, :]
    h = jnp.maximum(tmp, 0.0)
    o_ref[0, 0] = jnp.sum(h * w2_ref[...][None, :, :], axis=2)


def kernel(occ, params):
    p = params
    b = occ.shape[0]
    f32 = jnp.float32

    xp = jnp.pad(occ, ((0, 0), (1, 1), (1, 1), (1, 1)))
    sh = [xp[:, a:a + _M, c:c + _M, e:e + _M].reshape(b, _N3)
          for a in range(3) for c in range(3) for e in range(3)]
    xsh = jnp.stack(sh, axis=1)
    w1 = p['conv1_w'].reshape(32, 27)
    b1 = p['conv1_b'].reshape(32, 1)
    nck = 8
    ck = _N3 // nck
    h1 = pl.pallas_call(
        _conv_mm_kernel,
        grid=(b, nck),
        in_specs=[pl.BlockSpec((1, 27, ck), lambda i, n: (i, 0, n)),
                  pl.BlockSpec((32, 27), lambda i, n: (0, 0)),
                  pl.BlockSpec((32, 1), lambda i, n: (0, 0))],
        out_specs=pl.BlockSpec((1, 32, ck), lambda i, n: (i, 0, n)),
        out_shape=jax.ShapeDtypeStruct((b, 32, _N3), f32),
    )(xsh, w1, b1)

    x2 = h1.reshape(b, 32, 32, 2, 32, 2, 32, 2)
    x2 = x2.transpose(0, 1, 3, 5, 7, 2, 4, 6).reshape(b, 256, 32768)
    w2 = p['conv2_w'].reshape(64, 256)
    b2 = p['conv2_b'].reshape(64, 1)
    nck2 = 4
    ck2 = 32768 // nck2
    h2 = pl.pallas_call(
        _conv_mm_kernel,
        grid=(b, nck2),
        in_specs=[pl.BlockSpec((1, 256, ck2), lambda i, n: (i, 0, n)),
                  pl.BlockSpec((64, 256), lambda i, n: (0, 0)),
                  pl.BlockSpec((64, 1), lambda i, n: (0, 0))],
        out_specs=pl.BlockSpec((1, 64, ck2), lambda i, n: (i, 0, n)),
        out_shape=jax.ShapeDtypeStruct((b, 64, 32768), f32),
    )(x2, w2, b2)

    x3 = h2.reshape(b, 64, 16, 2, 16, 2, 16, 2)
    x3 = x3.transpose(0, 1, 3, 5, 7, 2, 4, 6).reshape(b, 512, 4096)
    w3 = p['conv3_w'].reshape(128, 512)
    b3 = p['conv3_b'].reshape(128, 1)
    occb = occ.reshape(b, 16, 4, 16, 4, 16, 4)
    occb = occb.transpose(0, 1, 3, 5, 2, 4, 6).reshape(b, 4096, 64)
    pooled = pl.pallas_call(
        _conv3_pool_kernel,
        grid=(b,),
        in_specs=[pl.BlockSpec((1, 512, 4096), lambda i: (i, 0, 0)),
                  pl.BlockSpec((128, 512), lambda i: (0, 0)),
                  pl.BlockSpec((128, 1), lambda i: (0, 0)),
                  pl.BlockSpec((1, 4096, 64), lambda i: (i, 0, 0))],
        out_specs=pl.BlockSpec((1, 128, 1), lambda i: (i, 0, 0)),
        out_shape=jax.ShapeDtypeStruct((b, 128, 1), f32),
    )(x3, w3, b3, occb)

    flat = 3 * _FN * _LAT
    pooled_t = pooled.reshape(b, 128).T
    fcb = p['fc_b'].reshape(flat, 1)
    nrow = 8
    rck = flat // nrow
    feat_t = pl.pallas_call(
        _fc_kernel,
        grid=(nrow,),
        in_specs=[pl.BlockSpec((rck, 128), lambda r: (r, 0)),
                  pl.BlockSpec((128, b), lambda r: (0, 0)),
                  pl.BlockSpec((rck, 1), lambda r: (r, 0))],
        out_specs=pl.BlockSpec((rck, b), lambda r: (r, 0)),
        out_shape=jax.ShapeDtypeStruct((flat, b), f32),
    )(p['fc_w'], pooled_t, fcb)

    x5 = feat_t.T.reshape(b, 3 * _FN, _LAT)
    eps = jax.random.normal(jax.random.key(42), (b, 3, _FN, _LAT), f32)
    eps = eps.reshape(b, 3 * _FN, _LAT)
    amat, kl3 = pl.pallas_call(
        _latent_kernel,
        grid=(b,),
        in_specs=[pl.BlockSpec((1, 3 * _FN, _LAT), lambda i: (i, 0, 0)),
                  pl.BlockSpec((1, 3 * _FN, _LAT), lambda i: (i, 0, 0)),
                  pl.BlockSpec((_LAT, _LAT), lambda i: (0, 0)),
                  pl.BlockSpec((1, _LAT), lambda i: (0, 0)),
                  pl.BlockSpec((_LAT, _LAT), lambda i: (0, 0)),
                  pl.BlockSpec((1, _LAT), lambda i: (0, 0)),
                  pl.BlockSpec((_FN, _FN - 1), lambda i: (0, 0)),
                  pl.BlockSpec((1, _FN - 1), lambda i: (0, 0)),
                  pl.BlockSpec((3 * _FD, _M), lambda i: (0, 0))],
        out_specs=[pl.BlockSpec((1, 3, _M, _M), lambda i: (i, 0, 0, 0)),
                   pl.BlockSpec((1, 1, 1), lambda i: (i, 0, 0))],
        out_shape=[jax.ShapeDtypeStruct((b, 3, _M, _M), f32),
                   jax.ShapeDtypeStruct((b, 1, 1), f32)],
    )(x5, eps,
      p['mu_w'].T, p['mu_b'].reshape(1, _LAT),
      p['logvar_w'].T, p['logvar_b'].reshape(1, _LAT),
      p['delta_w'].T, p['delta_b'].reshape(1, _FN - 1),
      p['dec1_w'].T)

    logits4 = pl.pallas_call(
        _decode_kernel,
        grid=(b, _M),
        in_specs=[pl.BlockSpec((1, 3, _M, _M), lambda i, j: (i, 0, 0, 0)),
                  pl.BlockSpec((1, _M), lambda i, j: (0, 0)),
                  pl.BlockSpec((1, _M), lambda i, j: (0, 0))],
        out_specs=pl.BlockSpec((1, 1, _M, _M), lambda i, j: (i, j, 0, 0)),
        out_shape=jax.ShapeDtypeStruct((b, _M, _M, _M), f32),
    )(amat, p['dec1_b'].reshape(1, _M), p['dec2_w'])

    logits = logits4.reshape(b, _N3) + p['dec2_b'][0]
    kl = kl3.reshape(b)
    return (logits, kl)
